# baseline (speedup 1.0000x reference)
import jax
import jax.numpy as jnp
from jax.experimental import pallas as pl
from jax.experimental.pallas import tpu as pltpu

_BEV_H = 256
_BEV_W = 256
_TJ = 256
_NT = 256


def _splat_kernel(feat_ref, depth_ref, pe_ref, out_ref, A_ref, cnt_ref,
                  idx_ref, dw_ref, w_ref):
    cam = pl.program_id(1)
    ncam = pl.num_programs(1)
    j = pl.program_id(0)
    TJ = out_ref.shape[0]
    col = jax.lax.broadcasted_iota(jnp.int32, (TJ, 1), 0) + j * TJ

    @pl.when(cam == 0)
    def _init():
        out_ref[...] = jnp.zeros_like(out_ref)
        cnt_ref[...] = jnp.zeros_like(cnt_ref)

    A_ref[...] = jnp.zeros_like(A_ref)

    px = pe_ref[0, 0]
    py = pe_ref[0, 1]
    pz = pe_ref[0, 2]
    bx = ((px - (-51.2)) / 0.4).astype(jnp.int32)
    by = ((py - (-51.2)) / 0.4).astype(jnp.int32)
    valid = (
        (bx >= 0) & (bx < _BEV_W) & (by >= 0) & (by < _BEV_H)
        & (pz >= -5.0) & (pz <= 3.0)
    )
    w = valid.astype(jnp.float32)
    idx_ref[...] = by * _BEV_W + bx
    w_ref[...] = w
    dw_ref[...] = depth_ref[0] * w

    def body(d, carry):
        idx_d = idx_ref[pl.ds(d, 1), :]
        m = (col == idx_d).astype(jnp.float32)
        A_ref[...] += m * dw_ref[pl.ds(d, 1), :]
        cnt_ref[...] += jnp.sum(m * w_ref[pl.ds(d, 1), :], axis=1,
                                keepdims=True)
        return carry

    jax.lax.fori_loop(0, px.shape[0], body, 0)

    out_ref[...] += jax.lax.dot(
        A_ref[...], feat_ref[0],
        precision=jax.lax.Precision.HIGHEST,
        preferred_element_type=jnp.float32,
    )

    @pl.when(cam == ncam - 1)
    def _fin():
        out_ref[...] = out_ref[...] / (cnt_ref[...] + 1e-5)


def kernel(feat, depth, intrinsics, extrinsics, img_size):
    B, N, C, H, W = feat.shape
    D = depth.shape[2]
    HW = H * W

    depth_bins = jnp.linspace(1.0, 60.0, D)
    xs = jnp.linspace(0.0, W - 1, W)
    ys = jnp.linspace(0.0, H - 1, H)
    yy, xx = jnp.meshgrid(ys, xs, indexing="ij")
    xx = jnp.broadcast_to(xx[None], (D, H, W))
    yy = jnp.broadcast_to(yy[None], (D, H, W))
    dd = jnp.broadcast_to(depth_bins[:, None, None], (D, H, W))
    frustum = jnp.stack([xx, yy, dd], axis=-1).reshape(-1, 3)
    u, v, d = frustum[:, 0], frustum[:, 1], frustum[:, 2]
    uv1 = jnp.stack([u * d, v * d, d], axis=-1)

    img = img_size.astype(jnp.float32)
    scale_x = W / (img[1] / 16.0)
    scale_y = H / (img[0] / 16.0)
    K = intrinsics[0]
    K = K.at[:, 0, :].multiply(16.0 / scale_x)
    K = K.at[:, 1, :].multiply(16.0 / scale_y)
    E = extrinsics[0]
    Kinv = jnp.linalg.inv(K)

    pts_cam = jnp.einsum("nij,pj->npi", Kinv, uv1)
    pts_ego = jnp.einsum("nij,npj->npi", E[:, :3, :3], pts_cam)
    pts_ego = pts_ego + E[:, None, :3, 3]
    pe = pts_ego.reshape(N, D, HW, 3).transpose(0, 3, 1, 2)

    feat_t = feat[0].reshape(N, C, HW).transpose(0, 2, 1)
    depth_r = depth[0].reshape(N, D, HW)

    out = pl.pallas_call(
        _splat_kernel,
        grid=(_NT, N),
        in_specs=[
            pl.BlockSpec((1, HW, C), lambda j, cam: (cam, 0, 0)),
            pl.BlockSpec((1, D, HW), lambda j, cam: (cam, 0, 0)),
            pl.BlockSpec((1, 3, D, HW), lambda j, cam: (cam, 0, 0, 0)),
        ],
        out_specs=pl.BlockSpec((_TJ, C), lambda j, cam: (j, 0)),
        out_shape=jax.ShapeDtypeStruct((_NT * _TJ, C), jnp.float32),
        scratch_shapes=[
            pltpu.VMEM((_TJ, HW), jnp.float32),
            pltpu.VMEM((_TJ, 1), jnp.float32),
            pltpu.VMEM((D, HW), jnp.int32),
            pltpu.VMEM((D, HW), jnp.float32),
            pltpu.VMEM((D, HW), jnp.float32),
        ],
        compiler_params=pltpu.CompilerParams(
            dimension_semantics=("arbitrary", "arbitrary"),
        ),
    )(feat_t, depth_r, pe)

    bev = out[: _BEV_H * _BEV_W].reshape(_BEV_H, _BEV_W, C)
    return bev.transpose(2, 0, 1)[None]
